# Initial kernel scaffold; baseline (speedup 1.0000x reference)
import functools

import jax
import jax.numpy as jnp
from jax import lax
from jax.experimental import pallas as pl
from jax.experimental.pallas import tpu as pltpu
from jax.experimental.pallas import tpu_sc as plsc

NC = 2
NS = 16
L = 16
NW = NC * NS
CH = 128

DEG_W = 16


def _mesh():
    return plsc.VectorSubcoreMesh(core_axis_name="c", subcore_axis_name="s")


def _make_deg_kernel(NP, E_pad):
    cpw = E_pad // (NW * CH)
    rps = NP // NS

    @functools.partial(
        pl.kernel,
        out_type=jax.ShapeDtypeStruct((NC, NP, DEG_W), jnp.float32),
        mesh=_mesh(),
        scratch_types=[
            pltpu.VMEM((CH,), jnp.int32),
            pltpu.VMEM((CH, DEG_W), jnp.float32),
            pltpu.VMEM((rps, DEG_W), jnp.float32),
            pltpu.VMEM_SHARED((NP, DEG_W), jnp.float32),
        ],
    )
    def deg_kernel(dst_hbm, out_hbm, dst_v, ones_v, zbuf, acc):
        c = lax.axis_index("c")
        s = lax.axis_index("s")
        ones = jnp.ones((1, L), jnp.float32)
        zero = jnp.zeros((1, L), jnp.float32)

        @pl.loop(0, CH)
        def _(r):
            ones_v.at[pl.ds(r, 1), pl.ds(0, L)][...] = ones

        @pl.loop(0, rps)
        def _(r):
            zbuf.at[pl.ds(r, 1), pl.ds(0, L)][...] = zero

        pltpu.sync_copy(zbuf, acc.at[pl.ds(s * rps, rps)])
        plsc.subcore_barrier()

        w = c * NS + s

        @pl.loop(0, cpw)
        def _(j):
            base = (w * cpw + j) * CH
            pltpu.sync_copy(dst_hbm.at[pl.ds(base, CH)], dst_v)
            pltpu.sync_copy(ones_v, acc.at[dst_v], add=True)

        plsc.subcore_barrier()
        pltpu.sync_copy(acc.at[pl.ds(s * rps, rps)],
                        out_hbm.at[c].at[pl.ds(s * rps, rps)])

    return deg_kernel


def _make_agg_kernel(NP, E_pad, Dw):
    cpw = E_pad // (NW * CH)
    rps = NP // NS

    @functools.partial(
        pl.kernel,
        out_type=jax.ShapeDtypeStruct((NC, NP, Dw), jnp.float32),
        mesh=_mesh(),
        scratch_types=[
            pltpu.VMEM((CH,), jnp.int32),
            pltpu.VMEM((CH,), jnp.int32),
            pltpu.VMEM((CH, Dw), jnp.float32),
            pltpu.VMEM((rps, Dw), jnp.float32),
            pltpu.VMEM_SHARED((NP, Dw), jnp.float32),
        ],
    )
    def agg_kernel(table_hbm, src_hbm, dst_hbm, out_hbm,
                   src_v, dst_v, rows_v, zbuf, acc):
        c = lax.axis_index("c")
        s = lax.axis_index("s")
        zero = jnp.zeros((1, L), jnp.float32)

        @pl.loop(0, rps)
        def _(r):
            @pl.loop(0, Dw, step=L)
            def _(col):
                zbuf.at[pl.ds(r, 1), pl.ds(col, L)][...] = zero

        pltpu.sync_copy(zbuf, acc.at[pl.ds(s * rps, rps)])
        plsc.subcore_barrier()

        w = c * NS + s

        @pl.loop(0, cpw)
        def _(j):
            base = (w * cpw + j) * CH
            pltpu.sync_copy(src_hbm.at[pl.ds(base, CH)], src_v)
            pltpu.sync_copy(dst_hbm.at[pl.ds(base, CH)], dst_v)
            pltpu.sync_copy(table_hbm.at[src_v], rows_v)
            pltpu.sync_copy(rows_v, acc.at[dst_v], add=True)

        plsc.subcore_barrier()
        pltpu.sync_copy(acc.at[pl.ds(s * rps, rps)],
                        out_hbm.at[c].at[pl.ds(s * rps, rps)])

    return agg_kernel


def _tc_matmul(x_pad, W1, NP, H1):
    def body(x_ref, w_ref, o_ref):
        o_ref[...] = jnp.dot(x_ref[...], w_ref[...],
                             preferred_element_type=jnp.float32)

    return pl.pallas_call(
        body,
        out_shape=jax.ShapeDtypeStruct((NP, H1), jnp.float32),
    )(x_pad, W1)


def _tc_scale(degp, h1, NP, H1):
    def body(degp_ref, h1_ref, dis_ref, hh_ref):
        d = degp_ref[...]
        tot = jnp.sum(d[0] + d[1], axis=1) * (1.0 / DEG_W) + 1.0
        dis = lax.rsqrt(tot)[:, None]
        dis_ref[...] = dis
        hh_ref[...] = dis * h1_ref[...]

    return pl.pallas_call(
        body,
        out_shape=(
            jax.ShapeDtypeStruct((NP, 1), jnp.float32),
            jax.ShapeDtypeStruct((NP, H1), jnp.float32),
        ),
    )(degp, h1)


def _tc_mid(p1, hh1, dis, W2, b1, NP, H2):
    def body(p_ref, hh1_ref, dis_ref, w2_ref, b1_ref, hh2_ref):
        p = p_ref[...]
        pre = dis_ref[...] * (p[0] + p[1] + hh1_ref[...]) + b1_ref[...]
        h = jnp.maximum(pre, 0.0)
        h2 = jnp.dot(h, w2_ref[...], preferred_element_type=jnp.float32)
        hh2_ref[...] = dis_ref[...] * h2

    return pl.pallas_call(
        body,
        out_shape=jax.ShapeDtypeStruct((NP, H2), jnp.float32),
    )(p1, hh1, dis, W2, b1)


def _tc_out(p2, hh2, dis, b2, NP, H2):
    def body(p_ref, hh2_ref, dis_ref, b2_ref, o_ref):
        p = p_ref[...]
        o_ref[...] = dis_ref[...] * (p[0] + p[1] + hh2_ref[...]) + b2_ref[...]

    return pl.pallas_call(
        body,
        out_shape=jax.ShapeDtypeStruct((NP, H2), jnp.float32),
    )(p2, hh2, dis, b2)


def kernel(x, edge_index, W1, b1, W2, b2):
    N, D_IN = x.shape
    E = edge_index.shape[1]
    H1 = W1.shape[1]
    H2 = W2.shape[1]
    NP = N + L

    cpw = -(-E // (NW * CH))
    E_pad = NW * CH * cpw

    src = edge_index[0]
    dst = edge_index[1]
    pad_idx = jnp.full((E_pad - E,), N, jnp.int32)
    src_p = jnp.concatenate([src, pad_idx])
    dst_p = jnp.concatenate([dst, pad_idx])
    x_pad = jnp.concatenate([x, jnp.zeros((NP - N, D_IN), x.dtype)])

    deg_kernel = _make_deg_kernel(NP, E_pad)
    agg1_kernel = _make_agg_kernel(NP, E_pad, H1)
    agg2_kernel = _make_agg_kernel(NP, E_pad, H2)

    degp = deg_kernel(dst_p)
    h1 = _tc_matmul(x_pad, W1, NP, H1)
    dis, hh1 = _tc_scale(degp, h1, NP, H1)
    p1 = agg1_kernel(hh1, src_p, dst_p)
    hh2 = _tc_mid(p1, hh1, dis, W2, b1.reshape(1, H1), NP, H2)
    p2 = agg2_kernel(hh2, src_p, dst_p)
    out = _tc_out(p2, hh2, dis, b2.reshape(1, H2), NP, H2)
    return out[:N]


# speedup vs baseline: 19.7735x; 19.7735x over previous
import functools

import jax
import jax.numpy as jnp
from jax import lax
from jax.experimental import pallas as pl
from jax.experimental.pallas import tpu as pltpu
from jax.experimental.pallas import tpu_sc as plsc

NC = 2
NS = 16
L = 16
NW = NC * NS
CH = 128

DEG_W = 16


def _mesh():
    return plsc.VectorSubcoreMesh(core_axis_name="c", subcore_axis_name="s")


_SC_PARAMS = pltpu.CompilerParams(use_tc_tiling_on_sc=False)


def _make_deg_kernel(NP, E_pad):
    cpw = E_pad // (NW * CH)
    rps = NP // NS

    @functools.partial(
        pl.kernel,
        out_type=jax.ShapeDtypeStruct((NC, NP, DEG_W), jnp.float32),
        mesh=_mesh(),
        scratch_types=[
            pltpu.VMEM((CH,), jnp.int32),
            pltpu.VMEM((CH, DEG_W), jnp.float32),
            pltpu.VMEM((rps, DEG_W), jnp.float32),
            pltpu.VMEM_SHARED((NP, DEG_W), jnp.float32),
        ],
        compiler_params=_SC_PARAMS,
    )
    def deg_kernel(dst_hbm, out_hbm, dst_v, ones_v, zbuf, acc):
        c = lax.axis_index("c")
        s = lax.axis_index("s")
        ones = jnp.ones((1, L), jnp.float32)
        zero = jnp.zeros((1, L), jnp.float32)

        @pl.loop(0, CH)
        def _(r):
            ones_v.at[pl.ds(r, 1), pl.ds(0, L)][...] = ones

        @pl.loop(0, rps)
        def _(r):
            zbuf.at[pl.ds(r, 1), pl.ds(0, L)][...] = zero

        row0 = pl.multiple_of(s * rps, 8)
        pltpu.sync_copy(zbuf, acc.at[pl.ds(row0, rps)])
        plsc.subcore_barrier()

        w = c * NS + s

        @pl.loop(0, cpw)
        def _(j):
            base = pl.multiple_of((w * cpw + j) * CH, CH)
            pltpu.sync_copy(dst_hbm.at[pl.ds(base, CH)], dst_v)
            pltpu.sync_copy(ones_v, acc.at[dst_v], add=True)

        plsc.subcore_barrier()
        pltpu.sync_copy(acc.at[pl.ds(row0, rps)],
                        out_hbm.at[c].at[pl.ds(row0, rps)])

    return deg_kernel


def _make_agg_kernel(NP, E_pad, Dw):
    cpw = E_pad // (NW * CH)
    rps = NP // NS

    @functools.partial(
        pl.kernel,
        out_type=jax.ShapeDtypeStruct((NC, NP, Dw), jnp.float32),
        mesh=_mesh(),
        scratch_types=[
            pltpu.VMEM((CH,), jnp.int32),
            pltpu.VMEM((CH,), jnp.int32),
            pltpu.VMEM((CH, Dw), jnp.float32),
            pltpu.VMEM((rps, Dw), jnp.float32),
            pltpu.VMEM_SHARED((NP, Dw), jnp.float32),
        ],
        compiler_params=_SC_PARAMS,
    )
    def agg_kernel(table_hbm, src_hbm, dst_hbm, out_hbm,
                   src_v, dst_v, rows_v, zbuf, acc):
        c = lax.axis_index("c")
        s = lax.axis_index("s")
        zero = jnp.zeros((1, L), jnp.float32)

        @pl.loop(0, rps)
        def _(r):
            @pl.loop(0, Dw, step=L)
            def _(col):
                zbuf.at[pl.ds(r, 1), pl.ds(col, L)][...] = zero

        row0 = pl.multiple_of(s * rps, 8)
        pltpu.sync_copy(zbuf, acc.at[pl.ds(row0, rps)])
        plsc.subcore_barrier()

        w = c * NS + s

        @pl.loop(0, cpw)
        def _(j):
            base = pl.multiple_of((w * cpw + j) * CH, CH)
            pltpu.sync_copy(src_hbm.at[pl.ds(base, CH)], src_v)
            pltpu.sync_copy(dst_hbm.at[pl.ds(base, CH)], dst_v)
            pltpu.sync_copy(table_hbm.at[src_v], rows_v)
            pltpu.sync_copy(rows_v, acc.at[dst_v], add=True)

        plsc.subcore_barrier()
        pltpu.sync_copy(acc.at[pl.ds(row0, rps)],
                        out_hbm.at[c].at[pl.ds(row0, rps)])

    return agg_kernel


def _tc_matmul(x_pad, W1, NP, H1):
    def body(x_ref, w_ref, o_ref):
        o_ref[...] = jnp.dot(x_ref[...], w_ref[...],
                             preferred_element_type=jnp.float32)

    return pl.pallas_call(
        body,
        out_shape=jax.ShapeDtypeStruct((NP, H1), jnp.float32),
    )(x_pad, W1)


def _tc_scale(degp, h1, NP, H1):
    def body(degp_ref, h1_ref, dis_ref, hh_ref):
        d = degp_ref[...]
        tot = jnp.sum(d[0] + d[1], axis=1) * (1.0 / DEG_W) + 1.0
        dis = lax.rsqrt(tot)[:, None]
        dis_ref[...] = dis
        hh_ref[...] = dis * h1_ref[...]

    return pl.pallas_call(
        body,
        out_shape=(
            jax.ShapeDtypeStruct((NP, 1), jnp.float32),
            jax.ShapeDtypeStruct((NP, H1), jnp.float32),
        ),
    )(degp, h1)


def _tc_mid(p1, hh1, dis, W2, b1, NP, H2):
    def body(p_ref, hh1_ref, dis_ref, w2_ref, b1_ref, hh2_ref):
        p = p_ref[...]
        pre = dis_ref[...] * (p[0] + p[1] + hh1_ref[...]) + b1_ref[...]
        h = jnp.maximum(pre, 0.0)
        h2 = jnp.dot(h, w2_ref[...], preferred_element_type=jnp.float32)
        hh2_ref[...] = dis_ref[...] * h2

    return pl.pallas_call(
        body,
        out_shape=jax.ShapeDtypeStruct((NP, H2), jnp.float32),
    )(p1, hh1, dis, W2, b1)


def _tc_out(p2, hh2, dis, b2, NP, H2):
    def body(p_ref, hh2_ref, dis_ref, b2_ref, o_ref):
        p = p_ref[...]
        o_ref[...] = dis_ref[...] * (p[0] + p[1] + hh2_ref[...]) + b2_ref[...]

    return pl.pallas_call(
        body,
        out_shape=jax.ShapeDtypeStruct((NP, H2), jnp.float32),
    )(p2, hh2, dis, b2)


def kernel(x, edge_index, W1, b1, W2, b2):
    N, D_IN = x.shape
    E = edge_index.shape[1]
    H1 = W1.shape[1]
    H2 = W2.shape[1]
    NP = -(-N // 128) * 128

    cpw = -(-E // (NW * CH))
    E_pad = NW * CH * cpw

    src = edge_index[0]
    dst = edge_index[1]
    pad_idx = jnp.full((E_pad - E,), N, jnp.int32)
    src_p = jnp.concatenate([src, pad_idx])
    dst_p = jnp.concatenate([dst, pad_idx])
    x_pad = jnp.concatenate([x, jnp.zeros((NP - N, D_IN), x.dtype)])

    deg_kernel = _make_deg_kernel(NP, E_pad)
    agg1_kernel = _make_agg_kernel(NP, E_pad, H1)
    agg2_kernel = _make_agg_kernel(NP, E_pad, H2)

    degp = deg_kernel(dst_p)
    h1 = _tc_matmul(x_pad, W1, NP, H1)
    dis, hh1 = _tc_scale(degp, h1, NP, H1)
    p1 = agg1_kernel(hh1, src_p, dst_p)
    hh2 = _tc_mid(p1, hh1, dis, W2, b1.reshape(1, H1), NP, H2)
    p2 = agg2_kernel(hh2, src_p, dst_p)
    out = _tc_out(p2, hh2, dis, b2.reshape(1, H2), NP, H2)
    return out[:N]


# speedup vs baseline: 33.6446x; 1.7015x over previous
import functools

import jax
import jax.numpy as jnp
from jax import lax
from jax.experimental import pallas as pl
from jax.experimental.pallas import tpu as pltpu
from jax.experimental.pallas import tpu_sc as plsc

NC = 2
NS = 16
L = 16
NW = NC * NS
CH = 128
G = 8

DEG_W = 16


def _mesh():
    return plsc.VectorSubcoreMesh(core_axis_name="c", subcore_axis_name="s")


_SC_PARAMS = pltpu.CompilerParams(use_tc_tiling_on_sc=False)


def _zero_rows(zbuf, rps, Dw):
    zero = jnp.zeros((1, L), jnp.float32)

    @pl.loop(0, rps)
    def _(r):
        @pl.loop(0, Dw, step=L)
        def _(col):
            zbuf.at[pl.ds(r, 1), pl.ds(col, L)][...] = zero


def _make_deg_kernel(NP, cpw):
    rps = NP // NS

    @functools.partial(
        pl.kernel,
        out_type=jax.ShapeDtypeStruct((NC, NP, DEG_W), jnp.float32),
        mesh=_mesh(),
        scratch_types=[
            pltpu.VMEM((cpw, CH), jnp.int32),
            pltpu.VMEM((CH, DEG_W), jnp.float32),
            pltpu.VMEM((rps, DEG_W), jnp.float32),
            pltpu.VMEM_SHARED((NP, DEG_W), jnp.float32),
            pltpu.SemaphoreType.DMA,
        ],
        compiler_params=_SC_PARAMS,
    )
    def deg_kernel(dst_hbm, out_hbm, dst_v, ones_v, zbuf, acc, sem):
        c = lax.axis_index("c")
        s = lax.axis_index("s")
        ones = jnp.ones((1, L), jnp.float32)

        @pl.loop(0, CH)
        def _(r):
            ones_v.at[pl.ds(r, 1), pl.ds(0, L)][...] = ones

        _zero_rows(zbuf, rps, DEG_W)
        row0 = pl.multiple_of(s * rps, 8)
        pltpu.sync_copy(zbuf, acc.at[pl.ds(row0, rps)])
        plsc.subcore_barrier()

        w = c * NS + s
        pltpu.sync_copy(dst_hbm.at[w], dst_v)

        @pl.loop(0, cpw)
        def _(j):
            pltpu.async_copy(ones_v, acc.at[dst_v.at[j]], sem, add=True)

        @pl.loop(0, cpw)
        def _(j):
            pltpu.make_async_copy(ones_v, acc.at[dst_v.at[0]], sem).wait()

        plsc.subcore_barrier()
        pltpu.sync_copy(acc.at[pl.ds(row0, rps)],
                        out_hbm.at[c].at[pl.ds(row0, rps)])

    return deg_kernel


def _make_agg_kernel(NP, cpw, Dw):
    rps = NP // NS
    S = cpw // G

    @functools.partial(
        pl.kernel,
        out_type=jax.ShapeDtypeStruct((NC, NP, Dw), jnp.float32),
        mesh=_mesh(),
        scratch_types=[
            pltpu.VMEM((cpw, CH), jnp.int32),
            pltpu.VMEM((cpw, CH), jnp.int32),
            pltpu.VMEM((G * CH, Dw), jnp.float32),
            pltpu.VMEM((G * CH, Dw), jnp.float32),
            pltpu.VMEM((rps, Dw), jnp.float32),
            pltpu.VMEM_SHARED((NP, Dw), jnp.float32),
            pltpu.SemaphoreType.DMA,
            pltpu.SemaphoreType.DMA,
            pltpu.SemaphoreType.DMA,
            pltpu.SemaphoreType.DMA,
        ],
        compiler_params=_SC_PARAMS,
    )
    def agg_kernel(table_hbm, src_hbm, dst_hbm, out_hbm,
                   src_v, dst_v, rows_a, rows_b, zbuf, acc,
                   sem_ga, sem_gb, sem_sa, sem_sb):
        c = lax.axis_index("c")
        s = lax.axis_index("s")

        _zero_rows(zbuf, rps, Dw)
        row0 = pl.multiple_of(s * rps, 8)
        pltpu.sync_copy(zbuf, acc.at[pl.ds(row0, rps)])
        plsc.subcore_barrier()

        w = c * NS + s
        pltpu.sync_copy(src_hbm.at[w], src_v)
        pltpu.sync_copy(dst_hbm.at[w], dst_v)

        def fire_g(sc_idx, rows, sem):
            for g in range(G):
                pltpu.async_copy(table_hbm.at[src_v.at[sc_idx * G + g]],
                                 rows.at[pl.ds(g * CH, CH)], sem)

        def drain_g(rows, sem):
            for g in range(G):
                pltpu.make_async_copy(table_hbm.at[src_v.at[0]],
                                      rows.at[pl.ds(g * CH, CH)], sem).wait()

        def fire_s(sc_idx, rows, sem):
            for g in range(G):
                pltpu.async_copy(rows.at[pl.ds(g * CH, CH)],
                                 acc.at[dst_v.at[sc_idx * G + g]], sem,
                                 add=True)

        def drain_s(rows, sem):
            for g in range(G):
                pltpu.make_async_copy(rows.at[pl.ds(g * CH, CH)],
                                      acc.at[dst_v.at[0]], sem).wait()

        fire_g(0, rows_a, sem_ga)

        @pl.loop(0, S // 2)
        def _(t):
            s0 = t * 2
            s1 = s0 + 1
            drain_g(rows_a, sem_ga)

            @pl.when(t > 0)
            def _():
                drain_s(rows_b, sem_sb)

            fire_g(s1, rows_b, sem_gb)
            fire_s(s0, rows_a, sem_sa)
            drain_g(rows_b, sem_gb)
            drain_s(rows_a, sem_sa)

            @pl.when(t + 1 < S // 2)
            def _():
                fire_g(s0 + 2, rows_a, sem_ga)

            fire_s(s1, rows_b, sem_sb)

        drain_s(rows_b, sem_sb)
        plsc.subcore_barrier()
        pltpu.sync_copy(acc.at[pl.ds(row0, rps)],
                        out_hbm.at[c].at[pl.ds(row0, rps)])

    return agg_kernel


def _tc_matmul(x_pad, W1, NP, H1):
    def body(x_ref, w_ref, o_ref):
        o_ref[...] = jnp.dot(x_ref[...], w_ref[...],
                             preferred_element_type=jnp.float32)

    return pl.pallas_call(
        body,
        out_shape=jax.ShapeDtypeStruct((NP, H1), jnp.float32),
    )(x_pad, W1)


def _tc_scale(degp, h1, NP, H1):
    def body(degp_ref, h1_ref, dis_ref, hh_ref):
        d = degp_ref[...]
        tot = jnp.sum(d[0] + d[1], axis=1) * (1.0 / DEG_W) + 1.0
        dis = lax.rsqrt(tot)[:, None]
        dis_ref[...] = dis
        hh_ref[...] = dis * h1_ref[...]

    return pl.pallas_call(
        body,
        out_shape=(
            jax.ShapeDtypeStruct((NP, 1), jnp.float32),
            jax.ShapeDtypeStruct((NP, H1), jnp.float32),
        ),
    )(degp, h1)


def _tc_mid(p1, hh1, dis, W2, b1, NP, H2):
    def body(p_ref, hh1_ref, dis_ref, w2_ref, b1_ref, hh2_ref):
        p = p_ref[...]
        pre = dis_ref[...] * (p[0] + p[1] + hh1_ref[...]) + b1_ref[...]
        h = jnp.maximum(pre, 0.0)
        h2 = jnp.dot(h, w2_ref[...], preferred_element_type=jnp.float32)
        hh2_ref[...] = dis_ref[...] * h2

    return pl.pallas_call(
        body,
        out_shape=jax.ShapeDtypeStruct((NP, H2), jnp.float32),
    )(p1, hh1, dis, W2, b1)


def _tc_out(p2, hh2, dis, b2, NP, H2):
    def body(p_ref, hh2_ref, dis_ref, b2_ref, o_ref):
        p = p_ref[...]
        o_ref[...] = dis_ref[...] * (p[0] + p[1] + hh2_ref[...]) + b2_ref[...]

    return pl.pallas_call(
        body,
        out_shape=jax.ShapeDtypeStruct((NP, H2), jnp.float32),
    )(p2, hh2, dis, b2)


def kernel(x, edge_index, W1, b1, W2, b2):
    N, D_IN = x.shape
    E = edge_index.shape[1]
    H1 = W1.shape[1]
    H2 = W2.shape[1]
    NP = -(-N // 128) * 128

    cpw = -(-E // (NW * CH * 2 * G)) * 2 * G
    E_pad = NW * CH * cpw

    src = edge_index[0]
    dst = edge_index[1]
    pad_idx = jnp.full((E_pad - E,), N, jnp.int32)
    src_p = jnp.concatenate([src, pad_idx]).reshape(NW, cpw, CH)
    dst_p = jnp.concatenate([dst, pad_idx]).reshape(NW, cpw, CH)
    x_pad = jnp.concatenate([x, jnp.zeros((NP - N, D_IN), x.dtype)])

    deg_kernel = _make_deg_kernel(NP, cpw)
    agg1_kernel = _make_agg_kernel(NP, cpw, H1)
    agg2_kernel = _make_agg_kernel(NP, cpw, H2)

    degp = deg_kernel(dst_p)
    h1 = _tc_matmul(x_pad, W1, NP, H1)
    dis, hh1 = _tc_scale(degp, h1, NP, H1)
    p1 = agg1_kernel(hh1, src_p, dst_p)
    hh2 = _tc_mid(p1, hh1, dis, W2, b1.reshape(1, H1), NP, H2)
    p2 = agg2_kernel(hh2, src_p, dst_p)
    out = _tc_out(p2, hh2, dis, b2.reshape(1, H2), NP, H2)
    return out[:N]
